# Initial kernel scaffold; baseline (speedup 1.0000x reference)
import jax
import jax.numpy as jnp
from jax.experimental import pallas as pl


def kernel(x, edge_index, edge_attr, batch, W1n, W1e, W1s, b1, W2n, W2e, W2s, b2, We1, be1, We2, be2):
    raise NotImplementedError("write your pallas kernel here")


# speedup vs baseline: 1.4528x; 1.4528x over previous
import functools

import jax
import jax.numpy as jnp
from jax import lax
from jax.experimental import pallas as pl
from jax.experimental.pallas import tpu as pltpu
from jax.experimental.pallas import tpu_sc as plsc

NC = 2
NS = 16
LANES = 16


def _pick_row_block(n, target=1024):
    for r in range(min(n, target), 7, -8):
        if n % r == 0:
            return r
    return n


def _tc_pre_node(x, W1n, W1s):
    N, D = x.shape
    DH = D // 2
    R = _pick_row_block(N)
    NB = N // R

    def body(x_ref, w1n_ref, w1s_ref, t_ref, s_ref):
        a = x_ref[...]
        t_ref[...] = jnp.dot(a, w1n_ref[...], preferred_element_type=jnp.float32)
        s_ref[...] = jnp.dot(a, w1s_ref[...], preferred_element_type=jnp.float32)

    return pl.pallas_call(
        body,
        grid=(NB, 2),
        in_specs=[
            pl.BlockSpec((R, D), lambda i, j: (i, 0)),
            pl.BlockSpec((D, DH), lambda i, j: (0, j)),
            pl.BlockSpec((D, DH), lambda i, j: (0, j)),
        ],
        out_specs=[
            pl.BlockSpec((R, DH), lambda i, j: (j * NB + i, 0)),
            pl.BlockSpec((R, DH), lambda i, j: (i, j)),
        ],
        out_shape=[
            jax.ShapeDtypeStruct((2 * N, DH), jnp.float32),
            jax.ShapeDtypeStruct((N, D), jnp.float32),
        ],
    )(x, W1n, W1s)


def _tc_pre_edge(ea, W1e, W2e):
    E, DE = ea.shape
    D = W1e.shape[1]
    DH = D // 2
    R = _pick_row_block(E, 8000)
    EB = E // R

    def body(ea_ref, w1_ref, w2_ref, o1_ref, o2_ref):
        a = ea_ref[...]
        o1_ref[...] = jnp.dot(a, w1_ref[...], preferred_element_type=jnp.float32)
        o2_ref[...] = jnp.dot(a, w2_ref[...], preferred_element_type=jnp.float32)

    return pl.pallas_call(
        body,
        grid=(EB, 2),
        in_specs=[
            pl.BlockSpec((R, DE), lambda i, j: (i, 0)),
            pl.BlockSpec((DE, DH), lambda i, j: (0, j)),
            pl.BlockSpec((DE, DH), lambda i, j: (0, j)),
        ],
        out_specs=[
            pl.BlockSpec((R, DH), lambda i, j: (j * EB + i, 0)),
            pl.BlockSpec((R, DH), lambda i, j: (j * EB + i, 0)),
        ],
        out_shape=[
            jax.ShapeDtypeStruct((2 * E, DH), jnp.float32),
            jax.ShapeDtypeStruct((2 * E, DH), jnp.float32),
        ],
    )(ea, W1e, W2e)


def _tc_mid(agg, skip, b, Wn, Ws):
    N, D = skip.shape
    DH = D // 2
    R = _pick_row_block(N)
    NB = N // R

    def body(lo_ref, hi_ref, skip_ref, b_ref, wn_ref, ws_ref, t_ref, s_ref):
        h = jnp.concatenate([lo_ref[...], hi_ref[...]], axis=1)
        h = jnp.maximum(h + skip_ref[...] + b_ref[...], 0.0)
        t_ref[...] = jnp.dot(h, wn_ref[...], preferred_element_type=jnp.float32)
        s_ref[...] = jnp.dot(h, ws_ref[...], preferred_element_type=jnp.float32)

    return pl.pallas_call(
        body,
        grid=(NB, 2),
        in_specs=[
            pl.BlockSpec((R, DH), lambda i, j: (i, 0)),
            pl.BlockSpec((R, DH), lambda i, j: (NB + i, 0)),
            pl.BlockSpec((R, D), lambda i, j: (i, 0)),
            pl.BlockSpec((1, D), lambda i, j: (0, 0)),
            pl.BlockSpec((D, DH), lambda i, j: (0, j)),
            pl.BlockSpec((D, DH), lambda i, j: (0, j)),
        ],
        out_specs=[
            pl.BlockSpec((R, DH), lambda i, j: (j * NB + i, 0)),
            pl.BlockSpec((R, DH), lambda i, j: (i, j)),
        ],
        out_shape=[
            jax.ShapeDtypeStruct((2 * N, DH), jnp.float32),
            jax.ShapeDtypeStruct((N, D), jnp.float32),
        ],
    )(agg, agg, skip, b, Wn, Ws)


def _tc_emb(agg, skip, b, We1, be1):
    N, D = skip.shape
    DH = D // 2
    R = _pick_row_block(N)
    NB = N // R

    def body(lo_ref, hi_ref, skip_ref, b_ref, wa_ref, wb_ref, be1_ref, a_ref, b2_ref):
        h = jnp.concatenate([lo_ref[...], hi_ref[...]], axis=1)
        h = jnp.maximum(h + skip_ref[...] + b_ref[...], 0.0)
        a_ref[...] = (
            jnp.dot(h, wa_ref[...], preferred_element_type=jnp.float32) + be1_ref[...]
        )
        b2_ref[...] = jnp.dot(h, wb_ref[...], preferred_element_type=jnp.float32)

    return pl.pallas_call(
        body,
        grid=(NB, 2),
        in_specs=[
            pl.BlockSpec((R, DH), lambda i, j: (i, 0)),
            pl.BlockSpec((R, DH), lambda i, j: (NB + i, 0)),
            pl.BlockSpec((R, D), lambda i, j: (i, 0)),
            pl.BlockSpec((1, D), lambda i, j: (0, 0)),
            pl.BlockSpec((D, DH), lambda i, j: (0, j)),
            pl.BlockSpec((D, DH), lambda i, j: (1, j)),
            pl.BlockSpec((1, DH), lambda i, j: (0, j)),
        ],
        out_specs=[
            pl.BlockSpec((R, DH), lambda i, j: (i, j)),
            pl.BlockSpec((R, DH), lambda i, j: (i, j)),
        ],
        out_shape=[
            jax.ShapeDtypeStruct((N, D), jnp.float32),
            jax.ShapeDtypeStruct((N, D), jnp.float32),
        ],
    )(agg, agg, skip, b, We1, We1, be1)


def _tc_final(agg, skip, b):
    N, D = skip.shape
    DH = D // 2
    R = _pick_row_block(N)
    NB = N // R

    def body(agg_ref, skip_ref, b_ref, o_ref):
        o_ref[...] = jnp.maximum(agg_ref[...] + skip_ref[...] + b_ref[...], 0.0)

    return pl.pallas_call(
        body,
        grid=(NB, 2),
        in_specs=[
            pl.BlockSpec((R, DH), lambda i, j: (j * NB + i, 0)),
            pl.BlockSpec((R, DH), lambda i, j: (i, j)),
            pl.BlockSpec((1, DH), lambda i, j: (0, j)),
        ],
        out_specs=pl.BlockSpec((R, DH), lambda i, j: (i, j)),
        out_shape=jax.ShapeDtypeStruct((N, D), jnp.float32),
    )(agg, skip, b)


def _tc_logits(s16, be2):
    E, L = s16.shape
    R = _pick_row_block(E, 8000)
    EB = E // R

    def body(s_ref, b_ref, lo_ref, at_ref):
        v = jnp.sum(s_ref[...], axis=1, keepdims=True) + b_ref[...]
        lo_ref[...] = v
        at_ref[...] = jax.nn.sigmoid(v)

    return pl.pallas_call(
        body,
        grid=(EB,),
        in_specs=[
            pl.BlockSpec((R, L), lambda i: (i, 0)),
            pl.BlockSpec((1, 1), lambda i: (0, 0)),
        ],
        out_specs=[
            pl.BlockSpec((R, 1), lambda i: (i, 0)),
            pl.BlockSpec((R, 1), lambda i: (i, 0)),
        ],
        out_shape=[
            jax.ShapeDtypeStruct((E, 1), jnp.float32),
            jax.ShapeDtypeStruct((E, 1), jnp.float32),
        ],
    )(s16, be2)


def _pick_chunk(n, cap=128, mult=8):
    for k in range(cap - cap % mult, mult - 1, -mult):
        if n % k == 0:
            return k
    return mult


def _pick_writers(n):
    for wt in range(NS, 0, -1):
        if n % wt == 0 and (n // wt) % 8 == 0:
            return wt, n // wt
    return 1, n


def _sc_msgpass(src, dst, table, ew, att, store_m):
    E = src.shape[0]
    twoN, DH = table.shape
    N = twoN // 2
    EPT = E // NS
    K = _pick_chunk(EPT, mult=LANES)
    NCH = EPT // K
    WT, RPT = _pick_writers(N)
    use_att = att is not None

    mesh = plsc.VectorSubcoreMesh(core_axis_name="c", subcore_axis_name="s")

    out_type = [jax.ShapeDtypeStruct((2 * N, DH), jnp.float32)]
    if store_m:
        out_type = [jax.ShapeDtypeStruct((2 * E, DH), jnp.float32)] + out_type

    scratch = [
        pltpu.VMEM((K,), jnp.int32),
        pltpu.VMEM((K,), jnp.int32),
        pltpu.VMEM((K, DH), jnp.float32),
        pltpu.VMEM((K, DH), jnp.float32),
        pltpu.VMEM((K,), jnp.float32),
        pltpu.VMEM_SHARED((N, DH), jnp.float32),
        pltpu.SemaphoreType.DMA,
    ]

    def body(*refs):
        i = 0
        src_hbm = refs[i]; i += 1
        dst_hbm = refs[i]; i += 1
        table_hbm = refs[i]; i += 1
        ew_hbm = refs[i]; i += 1
        if use_att:
            att_hbm = refs[i]; i += 1
        z_hbm = refs[i]; i += 1
        if store_m:
            m_hbm = refs[i]; i += 1
        agg_hbm = refs[i]; i += 1
        idx_v, dst_v, rows_v, ew_v, att_v, acc, sem = refs[i:]

        cid = lax.axis_index("c")
        sid = lax.axis_index("s")

        @pl.when(sid < WT)
        def _():
            pltpu.sync_copy(z_hbm, acc.at[pl.ds(sid * RPT, RPT)])

        plsc.subcore_barrier()

        tile_base = sid * EPT
        row_off = cid * N
        e_off = cid * E

        def chunk(g, carry):
            base = tile_base + g * K
            pltpu.sync_copy(src_hbm.at[pl.ds(base, K)], idx_v)
            pltpu.sync_copy(dst_hbm.at[pl.ds(base, K)], dst_v)
            for j in range(K // LANES):
                sl = pl.ds(j * LANES, LANES)
                idx_v[sl] = idx_v[sl] + row_off
            pltpu.async_copy(table_hbm.at[idx_v], rows_v, sem).wait()
            pltpu.sync_copy(ew_hbm.at[pl.ds(e_off + base, K)], ew_v)
            if use_att:
                pltpu.sync_copy(att_hbm.at[pl.ds(base, K)], att_v)

            def group(g2, c2):
                e0 = g2 * LANES
                if use_att:
                    att16 = att_v[pl.ds(e0, LANES)]
                for l in range(LANES):
                    e = e0 + l
                    for j in range(DH // LANES):
                        sl = pl.ds(j * LANES, LANES)
                        v = jnp.maximum(rows_v[e, sl] + ew_v[e, sl], 0.0)
                        if use_att:
                            v = v * att16[l]
                        rows_v[e, sl] = v
                return c2

            lax.fori_loop(0, K // LANES, group, 0)
            if store_m:
                pltpu.sync_copy(rows_v, m_hbm.at[pl.ds(e_off + base, K)])
            pltpu.sync_copy(rows_v, acc.at[dst_v], add=True)
            return carry

        lax.fori_loop(0, NCH, chunk, 0)
        plsc.subcore_barrier()

        @pl.when(sid < WT)
        def _():
            pltpu.sync_copy(
                acc.at[pl.ds(sid * RPT, RPT)],
                agg_hbm.at[pl.ds(row_off + sid * RPT, RPT)],
            )

    zrows = jnp.zeros((RPT, DH), jnp.float32)
    args = [src, dst, table, ew]
    if use_att:
        args.append(att)
    args.append(zrows)

    out = pl.kernel(body, out_type=out_type, mesh=mesh, scratch_types=scratch)(*args)
    return tuple(out) if store_m else out[0]


def _sc_scale_agg(m, att, dst, N):
    twoE, DH = m.shape
    E = twoE // 2
    EPT = E // NS
    K = _pick_chunk(EPT, mult=LANES)
    NCH = EPT // K
    WT, RPT = _pick_writers(N)

    mesh = plsc.VectorSubcoreMesh(core_axis_name="c", subcore_axis_name="s")

    scratch = [
        pltpu.VMEM((K,), jnp.int32),
        pltpu.VMEM((K, DH), jnp.float32),
        pltpu.VMEM((K,), jnp.float32),
        pltpu.VMEM_SHARED((N, DH), jnp.float32),
        pltpu.SemaphoreType.DMA,
    ]

    def body(m_hbm, att_hbm, dst_hbm, z_hbm, agg_hbm, dst_v, rows_v, att_v, acc, sem):
        cid = lax.axis_index("c")
        sid = lax.axis_index("s")

        @pl.when(sid < WT)
        def _():
            pltpu.sync_copy(z_hbm, acc.at[pl.ds(sid * RPT, RPT)])

        plsc.subcore_barrier()
        tile_base = sid * EPT
        e_off = cid * E

        def chunk(g, carry):
            base = tile_base + g * K
            pltpu.sync_copy(dst_hbm.at[pl.ds(base, K)], dst_v)
            pltpu.sync_copy(m_hbm.at[pl.ds(e_off + base, K)], rows_v)
            pltpu.sync_copy(att_hbm.at[pl.ds(base, K)], att_v)

            def group(g2, c2):
                e0 = g2 * LANES
                att16 = att_v[pl.ds(e0, LANES)]
                for l in range(LANES):
                    e = e0 + l
                    for j in range(DH // LANES):
                        sl = pl.ds(j * LANES, LANES)
                        rows_v[e, sl] = rows_v[e, sl] * att16[l]
                return c2

            lax.fori_loop(0, K // LANES, group, 0)
            pltpu.sync_copy(rows_v, acc.at[dst_v], add=True)
            return carry

        lax.fori_loop(0, NCH, chunk, 0)
        plsc.subcore_barrier()

        @pl.when(sid < WT)
        def _():
            pltpu.sync_copy(
                acc.at[pl.ds(sid * RPT, RPT)],
                agg_hbm.at[pl.ds(cid * N + sid * RPT, RPT)],
            )

    zrows = jnp.zeros((RPT, DH), jnp.float32)
    out = pl.kernel(
        body,
        out_type=[jax.ShapeDtypeStruct((2 * N, DH), jnp.float32)],
        mesh=mesh,
        scratch_types=scratch,
    )(m, att, dst, zrows)
    return out[0]


def _sc_att(src, dst, embA, embB, we2):
    E = src.shape[0]
    N, D = embA.shape
    NW = NC * NS
    EPT = E // NW
    K = _pick_chunk(EPT, 64)
    NCH = EPT // K

    mesh = plsc.VectorSubcoreMesh(core_axis_name="c", subcore_axis_name="s")

    scratch = [
        pltpu.VMEM((K,), jnp.int32),
        pltpu.VMEM((K,), jnp.int32),
        pltpu.VMEM((K, D), jnp.float32),
        pltpu.VMEM((K, D), jnp.float32),
        pltpu.VMEM((K, LANES), jnp.float32),
        pltpu.VMEM((D,), jnp.float32),
        pltpu.SemaphoreType.DMA,
    ]

    def body(src_hbm, dst_hbm, a_hbm, b_hbm, w_hbm, s_hbm,
             sidx, didx, a_v, b_v, o_v, w_v, sem):
        cid = lax.axis_index("c")
        sid = lax.axis_index("s")
        wid = sid * NC + cid
        pltpu.sync_copy(w_hbm, w_v)
        tile_base = wid * EPT

        def chunk(g, carry):
            base = tile_base + g * K
            pltpu.sync_copy(src_hbm.at[pl.ds(base, K)], sidx)
            pltpu.sync_copy(dst_hbm.at[pl.ds(base, K)], didx)
            pltpu.async_copy(a_hbm.at[sidx], a_v, sem).wait()
            pltpu.async_copy(b_hbm.at[didx], b_v, sem).wait()

            def edge(e, c2):
                acc = jnp.zeros((LANES,), jnp.float32)
                for j in range(D // LANES):
                    sl = pl.ds(j * LANES, LANES)
                    t = jnp.maximum(a_v[e, sl] + b_v[e, sl], 0.0)
                    acc = acc + t * w_v[sl]
                o_v[e, pl.ds(0, LANES)] = acc
                return c2

            lax.fori_loop(0, K, edge, 0)
            pltpu.sync_copy(o_v, s_hbm.at[pl.ds(base, K)])
            return carry

        lax.fori_loop(0, NCH, chunk, 0)

    out = pl.kernel(
        body,
        out_type=[jax.ShapeDtypeStruct((E, LANES), jnp.float32)],
        mesh=mesh,
        scratch_types=scratch,
    )(src, dst, embA, embB, we2)
    return out[0]


def kernel(x, edge_index, edge_attr, batch, W1n, W1e, W1s, b1,
           W2n, W2e, W2s, b2, We1, be1, We2, be2):
    N, D = x.shape
    E = edge_index.shape[1]
    src = edge_index[0]
    dst = edge_index[1]
    b1r = b1.reshape(1, D)
    b2r = b2.reshape(1, D)
    be1r = be1.reshape(1, D)
    we2v = We2.reshape(D)
    be2r = be2.reshape(1, 1)

    xW1n_t, xW1s = _tc_pre_node(x, W1n, W1s)
    eW1_t, eW2_t = _tc_pre_edge(edge_attr, W1e, W2e)

    m1, agg1 = _sc_msgpass(src, dst, xW1n_t, eW1_t, None, store_m=True)
    h1W2n_t, h1W2s = _tc_mid(agg1, xW1s, b1r, W2n, W2s)
    agg2 = _sc_msgpass(src, dst, h1W2n_t, eW2_t, None, store_m=False)
    embA, embB = _tc_emb(agg2, h1W2s, b2r, We1, be1r)
    s16 = _sc_att(src, dst, embA, embB, we2v)
    logits, att2 = _tc_logits(s16, be2r)
    att = att2.reshape(E)

    agg1p = _sc_scale_agg(m1, att, dst, N)
    h1pW2n_t, h1pW2s = _tc_mid(agg1p, xW1s, b1r, W2n, W2s)
    agg2p = _sc_msgpass(src, dst, h1pW2n_t, eW2_t, att, store_m=False)
    node_embeddings = _tc_final(agg2p, h1pW2s, b2r)

    return (logits, att2, node_embeddings)
